# baseline (speedup 1.0000x reference)
import functools

import jax
import jax.numpy as jnp
from jax import lax
from jax.experimental import pallas as pl
from jax.experimental.pallas import tpu as pltpu
from jax.experimental.pallas import tpu_sc as plsc

N_STRUCT = 10000
N_SEQ = 9998
N_EDGES = 320000
NPAD = 10240
D_IN = 128
DH = 256
HALF = 128
KMER = 3
BN = 1024
NW = 32
E_TILE = N_EDGES // 16
CH = 80
ITERS = E_TILE // CH
NB = 4
SLAB = NPAD // 16

_F32 = jnp.float32
_BF16 = jnp.bfloat16


def _dot(a, b):
    return jnp.dot(a.astype(_BF16), b.astype(_BF16),
                   preferred_element_type=_F32)


def _lin_gc_body(x_ref, wl_ref, b_ref, wg_ref, o0_ref, o1_ref):
    t = _dot(x_ref[...], wl_ref[...])
    t = jnp.maximum(t + b_ref[...], 0.0)
    o = _dot(t, wg_ref[...])
    o0_ref[...] = o[:, :HALF]
    o1_ref[...] = o[:, HALF:]


def _lin_gc(x, w_lin, b_lin, w_gc):
    n = x.shape[0]
    return pl.pallas_call(
        _lin_gc_body,
        grid=(n // BN,),
        in_specs=[
            pl.BlockSpec((BN, D_IN), lambda i: (i, 0)),
            pl.BlockSpec((D_IN, DH), lambda i: (0, 0)),
            pl.BlockSpec((1, DH), lambda i: (0, 0)),
            pl.BlockSpec((DH, DH), lambda i: (0, 0)),
        ],
        out_specs=[
            pl.BlockSpec((BN, HALF), lambda i: (i, 0)),
            pl.BlockSpec((BN, HALF), lambda i: (i, 0)),
        ],
        out_shape=[jax.ShapeDtypeStruct((n, HALF), _F32)] * 2,
    )(x, w_lin, b_lin.reshape(1, DH), w_gc)


def _relu_gc_body(h0_ref, h1_ref, wg_ref, o0_ref, o1_ref):
    t0 = jnp.maximum(h0_ref[...], 0.0)
    t1 = jnp.maximum(h1_ref[...], 0.0)
    wg = wg_ref[...]
    o = _dot(t0, wg[:HALF])
    o = o + _dot(t1, wg[HALF:])
    o0_ref[...] = o[:, :HALF]
    o1_ref[...] = o[:, HALF:]


def _relu_gc(h0, h1, w_gc):
    n = h0.shape[0]
    return pl.pallas_call(
        _relu_gc_body,
        grid=(n // BN,),
        in_specs=[
            pl.BlockSpec((BN, HALF), lambda i: (i, 0)),
            pl.BlockSpec((BN, HALF), lambda i: (i, 0)),
            pl.BlockSpec((DH, DH), lambda i: (0, 0)),
        ],
        out_specs=[
            pl.BlockSpec((BN, HALF), lambda i: (i, 0)),
            pl.BlockSpec((BN, HALF), lambda i: (i, 0)),
        ],
        out_shape=[jax.ShapeDtypeStruct((n, HALF), _F32)] * 2,
    )(h0, h1, w_gc)


def _spmm_sc(h0, h1, eidx):
    mesh = plsc.VectorSubcoreMesh(core_axis_name="c", subcore_axis_name="s")

    @functools.partial(
        pl.kernel,
        mesh=mesh,
        out_type=[jax.ShapeDtypeStruct((NPAD, HALF), _F32)] * 2,
        scratch_types=(
            [pltpu.VMEM((2, CH), jnp.int32)] * NB
            + [pltpu.VMEM((CH, HALF), _F32)] * NB
            + [pltpu.VMEM_SHARED((NPAD, HALF), _F32)]
            + [pltpu.SemaphoreType.DMA] * (3 * NB)
        ),
    )
    def spmm(h0_hbm, h1_hbm, idx_hbm, o0_hbm, o1_hbm, *scr):
        idxb = scr[0:NB]
        bufs = scr[NB:2 * NB]
        acc = scr[2 * NB]
        sg = scr[2 * NB + 1:2 * NB + 1 + NB]
        ss = scr[2 * NB + 1 + NB:2 * NB + 1 + 2 * NB]
        si = scr[2 * NB + 1 + 2 * NB:2 * NB + 1 + 3 * NB]

        c = lax.axis_index("c")
        s = lax.axis_index("s")

        zero16 = jnp.zeros((16,), _F32)

        def _zrow(i, carry):
            for j in range(HALF // 16):
                bufs[0][i, pl.ds(j * 16, 16)] = zero16
            return carry

        lax.fori_loop(0, CH, _zrow, 0)
        for k in range(SLAB // CH):
            pltpu.sync_copy(bufs[0], acc.at[pl.ds(s * SLAB + k * CH, CH)])
        plsc.subcore_barrier()

        def _idx_slice(i):
            return idx_hbm.at[s, i]

        def _half(h_hbm, o_hbm):
            pltpu.async_copy(_idx_slice(0), idxb[0], si[0])
            pltpu.async_copy(_idx_slice(1), idxb[1], si[1])
            pltpu.make_async_copy(_idx_slice(0), idxb[0], si[0]).wait()
            pltpu.async_copy(h_hbm.at[idxb[0].at[0]], bufs[0], sg[0])

            def _section(i, q, g1=True, gw=True, g2=True):
                q1 = (q + 1) % NB
                q2 = (q + 2) % NB
                pltpu.make_async_copy(h_hbm.at[idxb[q].at[0]], bufs[q],
                                      sg[q]).wait()
                pltpu.async_copy(bufs[q], acc.at[idxb[q].at[1]], ss[q],
                                 add=True)
                if g1:
                    pltpu.make_async_copy(_idx_slice(i + 1), idxb[q1],
                                          si[q1]).wait()
                    pltpu.async_copy(h_hbm.at[idxb[q1].at[0]], bufs[q1],
                                     sg[q1])
                if gw:
                    pltpu.make_async_copy(bufs[q2], acc.at[idxb[q2].at[1]],
                                          ss[q2]).wait()
                if g2:
                    pltpu.async_copy(_idx_slice(i + 2), idxb[q2], si[q2])

            _section(0, 0, gw=False)
            _section(1, 1, gw=False)
            _section(2, 2)
            _section(3, 3)

            def _quad(j, carry):
                i0 = j * NB
                for q in range(NB):
                    _section(i0 + q, q)
                return carry

            lax.fori_loop(1, (ITERS - 8) // NB + 1, _quad, 0)

            _section(244, 0)
            _section(245, 1)
            _section(246, 2)
            _section(247, 3)
            _section(248, 0, g2=False)
            _section(249, 1, g1=False, g2=False)
            for q in range(2):
                pltpu.make_async_copy(bufs[q], acc.at[idxb[q].at[1]],
                                      ss[q]).wait()
            plsc.subcore_barrier()
            pltpu.sync_copy(acc.at[pl.ds(s * SLAB, SLAB)],
                            o_hbm.at[pl.ds(s * SLAB, SLAB)])

        @pl.when(c == 0)
        def _():
            _half(h0_hbm, o0_hbm)

        @pl.when(c == 1)
        def _():
            _half(h1_hbm, o1_hbm)

    return spmm(h0, h1, eidx)


def _prep_body(sx0_ref, sx1_ref, qx0_ref, qx1_ref, wg_ref,
               ss_ref, sq_ref, sx_ref, res_ref, carry_ref):
    i = pl.program_id(0)
    wg = wg_ref[...]
    sxc = jnp.concatenate([jnp.maximum(sx0_ref[...], 0.0),
                           jnp.maximum(sx1_ref[...], 0.0)], axis=1)
    qxc = jnp.concatenate([jnp.maximum(qx0_ref[...], 0.0),
                           jnp.maximum(qx1_ref[...], 0.0)], axis=1)
    sx_ref[...] = sxc
    ss_ref[...] = _dot(sxc, wg)
    sq_ref[...] = _dot(qxc, wg)

    prev = jnp.where(i == 0, jnp.zeros((2, DH), _F32), carry_ref[...])
    r = qxc + jnp.concatenate([prev[1:], qxc[:-1]], 0)
    r = r + jnp.concatenate([prev, qxc[:-2]], 0)
    rows = lax.broadcasted_iota(jnp.int32, (BN, 1), 0) + i * BN
    cnt = jnp.minimum(rows, N_SEQ - 1) - jnp.maximum(rows - (KMER - 1), 0) + 1
    res_ref[...] = r / jnp.maximum(cnt, 1).astype(_F32)
    carry_ref[...] = qxc[-2:]


def _prep(sx0, sx1, qx0, qx1, w_gap):
    return pl.pallas_call(
        _prep_body,
        grid=(NPAD // BN,),
        in_specs=[
            pl.BlockSpec((BN, HALF), lambda i: (i, 0)),
            pl.BlockSpec((BN, HALF), lambda i: (i, 0)),
            pl.BlockSpec((BN, HALF), lambda i: (i, 0)),
            pl.BlockSpec((BN, HALF), lambda i: (i, 0)),
            pl.BlockSpec((DH, 1), lambda i: (0, 0)),
        ],
        out_specs=[
            pl.BlockSpec((BN, 1), lambda i: (i, 0)),
            pl.BlockSpec((BN, 1), lambda i: (i, 0)),
            pl.BlockSpec((BN, DH), lambda i: (i, 0)),
            pl.BlockSpec((BN, DH), lambda i: (i, 0)),
        ],
        out_shape=[
            jax.ShapeDtypeStruct((NPAD, 1), _F32),
            jax.ShapeDtypeStruct((NPAD, 1), _F32),
            jax.ShapeDtypeStruct((NPAD, DH), _F32),
            jax.ShapeDtypeStruct((NPAD, DH), _F32),
        ],
        scratch_shapes=[pltpu.VMEM((2, DH), _F32)],
    )(sx0, sx1, qx0, qx1, w_gap.reshape(DH, 1))


def _softmax_body(ss_ref, sq_ref, sa_ref, qa_ref):
    rows = lax.broadcasted_iota(jnp.int32, (NPAD, 1), 0)

    def att(sc, nvalid):
        valid = rows < nvalid
        m = jnp.max(jnp.where(valid, sc, -jnp.inf))
        e = jnp.where(valid, jnp.exp(sc - m), 0.0)
        return e / jnp.sum(e)

    sa_ref[...] = att(ss_ref[...], N_STRUCT)
    qa_ref[...] = att(sq_ref[...], N_SEQ)


def _softmax(ss, sq):
    return pl.pallas_call(
        _softmax_body,
        out_shape=[jax.ShapeDtypeStruct((NPAD, 1), _F32)] * 2,
    )(ss, sq)


def _head_body(sx_ref, res_ref, wl2_ref, bl2_ref, wf1_ref, bf1_ref,
               wf2_ref, bf2_ref, wf3_ref, bf3_ref, pred_ref, x_ref):
    wl2 = wl2_ref[...]
    x = _dot(sx_ref[...], wl2[:DH])
    x = x + _dot(res_ref[...], wl2[DH:])
    x = x + bl2_ref[...]
    x_ref[...] = x
    h = jnp.maximum(_dot(x, wf1_ref[...])
                    + bf1_ref[...], 0.0)
    h = jnp.maximum(_dot(h, wf2_ref[...])
                    + bf2_ref[...], 0.0)
    p = _dot(h, wf3_ref[...]) + bf3_ref[...]
    pred_ref[...] = jax.nn.sigmoid(p)


def _head(sx, res, w_lin2, b_lin2, w_f1, b_f1, w_f2, b_f2, w_f3, b_f3):
    return pl.pallas_call(
        _head_body,
        grid=(NPAD // BN,),
        in_specs=[
            pl.BlockSpec((BN, DH), lambda i: (i, 0)),
            pl.BlockSpec((BN, DH), lambda i: (i, 0)),
            pl.BlockSpec((2 * DH, DH), lambda i: (0, 0)),
            pl.BlockSpec((1, DH), lambda i: (0, 0)),
            pl.BlockSpec((DH, DH), lambda i: (0, 0)),
            pl.BlockSpec((1, DH), lambda i: (0, 0)),
            pl.BlockSpec((DH, HALF), lambda i: (0, 0)),
            pl.BlockSpec((1, HALF), lambda i: (0, 0)),
            pl.BlockSpec((HALF, 1), lambda i: (0, 0)),
            pl.BlockSpec((1, 1), lambda i: (0, 0)),
        ],
        out_specs=[
            pl.BlockSpec((BN, 1), lambda i: (i, 0)),
            pl.BlockSpec((BN, DH), lambda i: (i, 0)),
        ],
        out_shape=[
            jax.ShapeDtypeStruct((NPAD, 1), _F32),
            jax.ShapeDtypeStruct((NPAD, DH), _F32),
        ],
    )(sx, res, w_lin2, b_lin2.reshape(1, DH), w_f1, b_f1.reshape(1, DH),
      w_f2, b_f2.reshape(1, HALF), w_f3, b_f3.reshape(1, 1))


def kernel(struct_x, seq_x, struct_edge, seq_edge, W_lin, b_lin, W_gc1, W_gc2,
           w_gap, W_lin2, b_lin2, W_f1, b_f1, W_f2, b_f2, W_f3, b_f3):
    sx_pad = jnp.pad(struct_x, ((0, NPAD - N_STRUCT), (0, 0)))
    qx_pad = jnp.pad(seq_x, ((0, NPAD - N_SEQ), (0, 0)))
    def _pack(edge):
        return jnp.concatenate(
            [edge[1].reshape(16, ITERS, 1, CH), edge[0].reshape(16, ITERS, 1, CH)],
            axis=2)

    s_eidx = _pack(struct_edge)
    q_eidx = _pack(seq_edge)

    h0, h1 = _lin_gc(sx_pad, W_lin, b_lin, W_gc1)
    a0, a1 = _spmm_sc(h0, h1, s_eidx)
    h0, h1 = _relu_gc(a0, a1, W_gc2)
    sxf0, sxf1 = _spmm_sc(h0, h1, s_eidx)

    g0, g1 = _lin_gc(qx_pad, W_lin, b_lin, W_gc1)
    b0, b1 = _spmm_sc(g0, g1, q_eidx)
    g0, g1 = _relu_gc(b0, b1, W_gc2)
    qxf0, qxf1 = _spmm_sc(g0, g1, q_eidx)

    ss, sq, sx_full, residu = _prep(sxf0, sxf1, qxf0, qxf1, w_gap)
    satt, qatt = _softmax(ss, sq)
    pred, x = _head(sx_full, residu, W_lin2, b_lin2, W_f1, b_f1,
                    W_f2, b_f2, W_f3, b_f3)
    return (pred[:N_STRUCT], satt[:N_STRUCT, 0], qatt[:N_SEQ, 0], x[:N_STRUCT])


# speedup vs baseline: 1.0133x; 1.0133x over previous
import functools

import jax
import jax.numpy as jnp
from jax import lax
from jax.experimental import pallas as pl
from jax.experimental.pallas import tpu as pltpu
from jax.experimental.pallas import tpu_sc as plsc

N_STRUCT = 10000
N_SEQ = 9998
N_EDGES = 320000
NPAD = 10240
D_IN = 128
DH = 256
HALF = 128
KMER = 3
BN = 1024
NW = 32
E_TILE = N_EDGES // 16
CH = 80
ITERS = E_TILE // CH
NB = 4
SLAB = NPAD // 16

_F32 = jnp.float32
_BF16 = jnp.bfloat16


def _dot(a, b):
    return jnp.dot(a.astype(_BF16), b.astype(_BF16),
                   preferred_element_type=_F32)


def _lin_gc_body(x_ref, wl_ref, b_ref, wg_ref, o0_ref, o1_ref):
    t = _dot(x_ref[...], wl_ref[...])
    t = jnp.maximum(t + b_ref[...], 0.0)
    o = _dot(t, wg_ref[...])
    o0_ref[...] = o[:, :HALF]
    o1_ref[...] = o[:, HALF:]


def _lin_gc(x, w_lin, b_lin, w_gc):
    n = x.shape[0]
    return pl.pallas_call(
        _lin_gc_body,
        grid=(n // BN,),
        in_specs=[
            pl.BlockSpec((BN, D_IN), lambda i: (i, 0)),
            pl.BlockSpec((D_IN, DH), lambda i: (0, 0)),
            pl.BlockSpec((1, DH), lambda i: (0, 0)),
            pl.BlockSpec((DH, DH), lambda i: (0, 0)),
        ],
        out_specs=[
            pl.BlockSpec((BN, HALF), lambda i: (i, 0)),
            pl.BlockSpec((BN, HALF), lambda i: (i, 0)),
        ],
        out_shape=[jax.ShapeDtypeStruct((n, HALF), _F32)] * 2,
    )(x, w_lin, b_lin.reshape(1, DH), w_gc)


def _relu_gc_body(h0_ref, h1_ref, wg_ref, o0_ref, o1_ref):
    t0 = jnp.maximum(h0_ref[...], 0.0)
    t1 = jnp.maximum(h1_ref[...], 0.0)
    wg = wg_ref[...]
    o = _dot(t0, wg[:HALF])
    o = o + _dot(t1, wg[HALF:])
    o0_ref[...] = o[:, :HALF]
    o1_ref[...] = o[:, HALF:]


def _relu_gc(h0, h1, w_gc):
    n = h0.shape[0]
    return pl.pallas_call(
        _relu_gc_body,
        grid=(n // BN,),
        in_specs=[
            pl.BlockSpec((BN, HALF), lambda i: (i, 0)),
            pl.BlockSpec((BN, HALF), lambda i: (i, 0)),
            pl.BlockSpec((DH, DH), lambda i: (0, 0)),
        ],
        out_specs=[
            pl.BlockSpec((BN, HALF), lambda i: (i, 0)),
            pl.BlockSpec((BN, HALF), lambda i: (i, 0)),
        ],
        out_shape=[jax.ShapeDtypeStruct((n, HALF), _F32)] * 2,
    )(h0, h1, w_gc)


def _spmm_sc(h0, h1, src2, dst2):
    mesh = plsc.VectorSubcoreMesh(core_axis_name="c", subcore_axis_name="s")

    @functools.partial(
        pl.kernel,
        mesh=mesh,
        out_type=[jax.ShapeDtypeStruct((NPAD, HALF), _F32)] * 2,
        scratch_types=(
            [pltpu.VMEM((CH,), jnp.int32)] * (2 * NB)
            + [pltpu.VMEM((CH, HALF), _F32)] * NB
            + [pltpu.VMEM_SHARED((NPAD, HALF), _F32)]
            + [pltpu.SemaphoreType.DMA] * (4 * NB)
        ),
    )
    def spmm(h0_hbm, h1_hbm, src_hbm, dst_hbm, o0_hbm, o1_hbm, *scr):
        sidx = scr[0:NB]
        didx = scr[NB:2 * NB]
        bufs = scr[2 * NB:3 * NB]
        acc = scr[3 * NB]
        sg = scr[3 * NB + 1:3 * NB + 1 + NB]
        ss = scr[3 * NB + 1 + NB:3 * NB + 1 + 2 * NB]
        sis = scr[3 * NB + 1 + 2 * NB:3 * NB + 1 + 3 * NB]
        sid = scr[3 * NB + 1 + 3 * NB:3 * NB + 1 + 4 * NB]

        c = lax.axis_index("c")
        s = lax.axis_index("s")
        ebase = s * E_TILE

        zero16 = jnp.zeros((16,), _F32)

        def _zrow(i, carry):
            for j in range(HALF // 16):
                bufs[0][i, pl.ds(j * 16, 16)] = zero16
            return carry

        lax.fori_loop(0, CH, _zrow, 0)
        for k in range(SLAB // CH):
            pltpu.sync_copy(bufs[0], acc.at[pl.ds(s * SLAB + k * CH, CH)])
        plsc.subcore_barrier()

        def _src_slice(i):
            return src_hbm.at[pl.ds(ebase + i * CH, CH)]

        def _dst_slice(i):
            return dst_hbm.at[pl.ds(ebase + i * CH, CH)]

        def _half(h_hbm, o_hbm):
            for q in range(NB):
                pltpu.async_copy(_src_slice(q), sidx[q], sis[q])
            pltpu.async_copy(_dst_slice(0), didx[0], sid[0])
            pltpu.async_copy(_dst_slice(1), didx[1], sid[1])
            pltpu.make_async_copy(_src_slice(0), sidx[0], sis[0]).wait()
            pltpu.async_copy(h_hbm.at[sidx[0]], bufs[0], sg[0])

            def _section(i, q, g1=True, gw=True, g2=True, g4=True):
                q1 = (q + 1) % NB
                q2 = (q + 2) % NB
                pltpu.make_async_copy(h_hbm.at[sidx[q]], bufs[q], sg[q]).wait()
                pltpu.make_async_copy(_dst_slice(i), didx[q], sid[q]).wait()
                pltpu.async_copy(bufs[q], acc.at[didx[q]], ss[q], add=True)
                if g1:
                    pltpu.make_async_copy(_src_slice(i + 1), sidx[q1],
                                          sis[q1]).wait()
                    pltpu.async_copy(h_hbm.at[sidx[q1]], bufs[q1], sg[q1])
                if gw:
                    pltpu.make_async_copy(bufs[q2], acc.at[didx[q2]],
                                          ss[q2]).wait()
                if g2:
                    pltpu.async_copy(_dst_slice(i + 2), didx[q2], sid[q2])
                if g4:
                    pltpu.async_copy(_src_slice(i + 4), sidx[q], sis[q])

            _section(0, 0, gw=False)
            _section(1, 1, gw=False)
            _section(2, 2)
            _section(3, 3)

            def _quad(j, carry):
                i0 = j * NB
                for q in range(NB):
                    _section(i0 + q, q)
                return carry

            lax.fori_loop(1, (ITERS - 8) // NB + 1, _quad, 0)

            _section(244, 0)
            _section(245, 1)
            _section(246, 2, g4=False)
            _section(247, 3, g4=False)
            _section(248, 0, g2=False, g4=False)
            _section(249, 1, g1=False, g2=False, g4=False)
            for q in range(2):
                pltpu.make_async_copy(bufs[q], acc.at[didx[q]], ss[q]).wait()
            plsc.subcore_barrier()
            pltpu.sync_copy(acc.at[pl.ds(s * SLAB, SLAB)],
                            o_hbm.at[pl.ds(s * SLAB, SLAB)])

        @pl.when(c == 0)
        def _():
            _half(h0_hbm, o0_hbm)

        @pl.when(c == 1)
        def _():
            _half(h1_hbm, o1_hbm)

    return spmm(h0, h1, src2, dst2)


def _prep_body(sx0_ref, sx1_ref, qx0_ref, qx1_ref, wg_ref,
               ss_ref, sq_ref, sx_ref, res_ref, carry_ref):
    i = pl.program_id(0)
    wg = wg_ref[...]
    sxc = jnp.concatenate([jnp.maximum(sx0_ref[...], 0.0),
                           jnp.maximum(sx1_ref[...], 0.0)], axis=1)
    qxc = jnp.concatenate([jnp.maximum(qx0_ref[...], 0.0),
                           jnp.maximum(qx1_ref[...], 0.0)], axis=1)
    sx_ref[...] = sxc
    ss_ref[...] = _dot(sxc, wg)
    sq_ref[...] = _dot(qxc, wg)

    prev = jnp.where(i == 0, jnp.zeros((2, DH), _F32), carry_ref[...])
    r = qxc + jnp.concatenate([prev[1:], qxc[:-1]], 0)
    r = r + jnp.concatenate([prev, qxc[:-2]], 0)
    rows = lax.broadcasted_iota(jnp.int32, (BN, 1), 0) + i * BN
    cnt = jnp.minimum(rows, N_SEQ - 1) - jnp.maximum(rows - (KMER - 1), 0) + 1
    res_ref[...] = r / jnp.maximum(cnt, 1).astype(_F32)
    carry_ref[...] = qxc[-2:]


def _prep(sx0, sx1, qx0, qx1, w_gap):
    return pl.pallas_call(
        _prep_body,
        grid=(NPAD // BN,),
        in_specs=[
            pl.BlockSpec((BN, HALF), lambda i: (i, 0)),
            pl.BlockSpec((BN, HALF), lambda i: (i, 0)),
            pl.BlockSpec((BN, HALF), lambda i: (i, 0)),
            pl.BlockSpec((BN, HALF), lambda i: (i, 0)),
            pl.BlockSpec((DH, 1), lambda i: (0, 0)),
        ],
        out_specs=[
            pl.BlockSpec((BN, 1), lambda i: (i, 0)),
            pl.BlockSpec((BN, 1), lambda i: (i, 0)),
            pl.BlockSpec((BN, DH), lambda i: (i, 0)),
            pl.BlockSpec((BN, DH), lambda i: (i, 0)),
        ],
        out_shape=[
            jax.ShapeDtypeStruct((NPAD, 1), _F32),
            jax.ShapeDtypeStruct((NPAD, 1), _F32),
            jax.ShapeDtypeStruct((NPAD, DH), _F32),
            jax.ShapeDtypeStruct((NPAD, DH), _F32),
        ],
        scratch_shapes=[pltpu.VMEM((2, DH), _F32)],
    )(sx0, sx1, qx0, qx1, w_gap.reshape(DH, 1))


def _softmax_body(ss_ref, sq_ref, sa_ref, qa_ref):
    rows = lax.broadcasted_iota(jnp.int32, (NPAD, 1), 0)

    def att(sc, nvalid):
        valid = rows < nvalid
        m = jnp.max(jnp.where(valid, sc, -jnp.inf))
        e = jnp.where(valid, jnp.exp(sc - m), 0.0)
        return e / jnp.sum(e)

    sa_ref[...] = att(ss_ref[...], N_STRUCT)
    qa_ref[...] = att(sq_ref[...], N_SEQ)


def _softmax(ss, sq):
    return pl.pallas_call(
        _softmax_body,
        out_shape=[jax.ShapeDtypeStruct((NPAD, 1), _F32)] * 2,
    )(ss, sq)


def _head_body(sx_ref, res_ref, wl2_ref, bl2_ref, wf1_ref, bf1_ref,
               wf2_ref, bf2_ref, wf3_ref, bf3_ref, pred_ref, x_ref):
    wl2 = wl2_ref[...]
    x = _dot(sx_ref[...], wl2[:DH])
    x = x + _dot(res_ref[...], wl2[DH:])
    x = x + bl2_ref[...]
    x_ref[...] = x
    h = jnp.maximum(_dot(x, wf1_ref[...])
                    + bf1_ref[...], 0.0)
    h = jnp.maximum(_dot(h, wf2_ref[...])
                    + bf2_ref[...], 0.0)
    p = _dot(h, wf3_ref[...]) + bf3_ref[...]
    pred_ref[...] = jax.nn.sigmoid(p)


def _head(sx, res, w_lin2, b_lin2, w_f1, b_f1, w_f2, b_f2, w_f3, b_f3):
    return pl.pallas_call(
        _head_body,
        grid=(NPAD // BN,),
        in_specs=[
            pl.BlockSpec((BN, DH), lambda i: (i, 0)),
            pl.BlockSpec((BN, DH), lambda i: (i, 0)),
            pl.BlockSpec((2 * DH, DH), lambda i: (0, 0)),
            pl.BlockSpec((1, DH), lambda i: (0, 0)),
            pl.BlockSpec((DH, DH), lambda i: (0, 0)),
            pl.BlockSpec((1, DH), lambda i: (0, 0)),
            pl.BlockSpec((DH, HALF), lambda i: (0, 0)),
            pl.BlockSpec((1, HALF), lambda i: (0, 0)),
            pl.BlockSpec((HALF, 1), lambda i: (0, 0)),
            pl.BlockSpec((1, 1), lambda i: (0, 0)),
        ],
        out_specs=[
            pl.BlockSpec((BN, 1), lambda i: (i, 0)),
            pl.BlockSpec((BN, DH), lambda i: (i, 0)),
        ],
        out_shape=[
            jax.ShapeDtypeStruct((NPAD, 1), _F32),
            jax.ShapeDtypeStruct((NPAD, DH), _F32),
        ],
    )(sx, res, w_lin2, b_lin2.reshape(1, DH), w_f1, b_f1.reshape(1, DH),
      w_f2, b_f2.reshape(1, HALF), w_f3, b_f3.reshape(1, 1))


def kernel(struct_x, seq_x, struct_edge, seq_edge, W_lin, b_lin, W_gc1, W_gc2,
           w_gap, W_lin2, b_lin2, W_f1, b_f1, W_f2, b_f2, W_f3, b_f3):
    sx_pad = jnp.pad(struct_x, ((0, NPAD - N_STRUCT), (0, 0)))
    qx_pad = jnp.pad(seq_x, ((0, NPAD - N_SEQ), (0, 0)))
    s_dst, s_src = struct_edge[0], struct_edge[1]
    q_dst, q_src = seq_edge[0], seq_edge[1]

    h0, h1 = _lin_gc(sx_pad, W_lin, b_lin, W_gc1)
    a0, a1 = _spmm_sc(h0, h1, s_src, s_dst)
    h0, h1 = _relu_gc(a0, a1, W_gc2)
    sxf0, sxf1 = _spmm_sc(h0, h1, s_src, s_dst)

    g0, g1 = _lin_gc(qx_pad, W_lin, b_lin, W_gc1)
    b0, b1 = _spmm_sc(g0, g1, q_src, q_dst)
    g0, g1 = _relu_gc(b0, b1, W_gc2)
    qxf0, qxf1 = _spmm_sc(g0, g1, q_src, q_dst)

    ss, sq, sx_full, residu = _prep(sxf0, sxf1, qxf0, qxf1, w_gap)
    satt, qatt = _softmax(ss, sq)
    pred, x = _head(sx_full, residu, W_lin2, b_lin2, W_f1, b_f1,
                    W_f2, b_f2, W_f3, b_f3)
    return (pred[:N_STRUCT], satt[:N_STRUCT, 0], qatt[:N_SEQ, 0], x[:N_STRUCT])
